# baseline (speedup 1.0000x reference)
import functools

import jax
import jax.numpy as jnp
from jax import lax
from jax.experimental import pallas as pl
from jax.experimental.pallas import tpu as pltpu
from jax.experimental.pallas import tpu_sc as plsc

B, N1, N2 = 8, 1024, 8192
C1, C2 = 256, 128
OUTC = 3 + C1 + C2
TILE = 512
NT = N2 // TILE
ROWS = B * N2

NC, NS = 2, 16
NW = NC * NS
ROWS_PER_W = ROWS // NW
CHUNK = 128
NCHUNK = ROWS_PER_W // CHUNK


def _argmin_body(xyz1t_ref, x1_ref, idx_ref):
    b = pl.program_id(0)
    xyz1t = xyz1t_ref[0]
    xyz2 = x1_ref[0][:, 0:3]
    cross = jnp.dot(xyz2, xyz1t, preferred_element_type=jnp.float32)
    x2sq = jnp.sum(xyz2 * xyz2, axis=1, keepdims=True)
    x1sq = jnp.sum(xyz1t * xyz1t, axis=0, keepdims=True)
    d = x2sq - 2.0 * cross + x1sq
    idx_ref[0, 0, :] = jnp.argmin(d, axis=1) + b * N1


def _sc_gather_body(f1_hbm, idx_hbm, out_hbm, idx_v, rows_v, sem):
    wid = lax.axis_index("s") * NC + lax.axis_index("c")
    base = wid * ROWS_PER_W

    def body(k, _):
        off = base + k * CHUNK
        pltpu.sync_copy(idx_hbm.at[pl.ds(off, CHUNK)], idx_v)
        pltpu.async_copy(f1_hbm.at[idx_v], rows_v, sem).wait()
        pltpu.sync_copy(rows_v, out_hbm.at[pl.ds(off, CHUNK)])
        return 0

    lax.fori_loop(0, NCHUNK, body, 0)


def _assemble_body(x1_ref, g_ref, out_ref):
    x1b = x1_ref[0]
    out_ref[0, :, 0:3] = x1b[:, 0:3]
    out_ref[0, :, 3:3 + C1] = g_ref[0]
    out_ref[0, :, 3 + C1:] = x1b[:, 3:]


def kernel(x0, x1):
    xyz1t = jnp.transpose(x0[:, :, 0:3], (0, 2, 1))
    f1_flat = jnp.reshape(x0[:, :, 3:], (B * N1, C1))

    idxq = pl.pallas_call(
        _argmin_body,
        grid=(B, NT),
        in_specs=[
            pl.BlockSpec((1, 3, N1), lambda b, t: (b, 0, 0)),
            pl.BlockSpec((1, TILE, 3 + C2), lambda b, t: (b, t, 0)),
        ],
        out_specs=pl.BlockSpec((1, 1, TILE), lambda b, t: (b * NT + t, 0, 0)),
        out_shape=jax.ShapeDtypeStruct((B * NT, 1, TILE), jnp.int32),
    )(xyz1t, x1)
    idx_flat = jnp.reshape(idxq, (ROWS,))

    mesh = plsc.VectorSubcoreMesh(core_axis_name="c", subcore_axis_name="s")
    gathered = pl.kernel(
        _sc_gather_body,
        out_type=jax.ShapeDtypeStruct((ROWS, C1), jnp.float32),
        mesh=mesh,
        scratch_types=[
            pltpu.VMEM((CHUNK,), jnp.int32),
            pltpu.VMEM((CHUNK, C1), jnp.float32),
            pltpu.SemaphoreType.DMA,
        ],
    )(f1_flat, idx_flat)

    g3 = jnp.reshape(gathered, (B, N2, C1))
    out = pl.pallas_call(
        _assemble_body,
        grid=(B, NT),
        in_specs=[
            pl.BlockSpec((1, TILE, 3 + C2), lambda b, t: (b, t, 0)),
            pl.BlockSpec((1, TILE, C1), lambda b, t: (b, t, 0)),
        ],
        out_specs=pl.BlockSpec((1, TILE, OUTC), lambda b, t: (b, t, 0)),
        out_shape=jax.ShapeDtypeStruct((B, N2, OUTC), jnp.float32),
    )(x1, g3)
    return (out, x1[:, :, 0:3])
